# baseline (speedup 1.0000x reference)
import functools

import numpy as np
import jax
import jax.numpy as jnp
from jax import lax
from jax.experimental import pallas as pl
from jax.experimental.pallas import tpu as pltpu
from jax.experimental.pallas import tpu_sc as plsc

_VOCAB, _EMBED, _OUT = 100000, 32, 128
_B, _L = 4096, 50
_NW = 32
_CH = 128
_RPW = _B // _NW
_IPW = _RPW * _L
_NCH = _IPW // _CH

_DST3 = (
    (np.arange(_IPW, dtype=np.int32) // _L).reshape(1, _NCH, _CH)
    + (np.arange(16, dtype=np.int32) * 2 * _RPW).reshape(16, 1, 1)
    + (np.arange(_NCH, dtype=np.int32) % 2 * _RPW).reshape(1, _NCH, 1)
)

_mesh = plsc.VectorSubcoreMesh(core_axis_name="c", subcore_axis_name="s")

_VPAD = 100096
_NCHUNK = 781
_TAIL_V = _NCHUNK * 128
_CPW = 25


@functools.partial(
    pl.kernel,
    mesh=_mesh,
    compiler_params=pltpu.CompilerParams(
        use_tc_tiling_on_sc=True, needs_layout_passes=False),
    out_type=jax.ShapeDtypeStruct((_VPAD * _EMBED,), jnp.float32),
    scratch_types=[
        pltpu.VMEM((_EMBED, 128), jnp.float32),
        pltpu.VMEM((128 * _EMBED,), jnp.float32),
        pltpu.VMEM((32 * _EMBED,), jnp.float32),
    ],
)
def _sc_relayout(tableT, tailflat, tflat, tbuf, obuf, tailbuf):
    cid = lax.axis_index("c")
    sid = lax.axis_index("s")
    wid = sid * 2 + cid

    sidx = lax.iota(jnp.int32, 16) * _EMBED

    def chunk(k, carry):
        j = wid + k * _NW

        @pl.when(j < _NCHUNK)
        def _():
            pltpu.sync_copy(tableT.at[:, pl.ds(j * 128, 128)], tbuf)
            for e in range(_EMBED):
                for g in range(8):
                    vec = tbuf[e, pl.ds(16 * g, 16)]
                    plsc.store_scatter(obuf, [sidx + (16 * g * _EMBED + e)],
                                       vec)
            pltpu.sync_copy(obuf, tflat.at[pl.ds(j * 128 * _EMBED,
                                                 128 * _EMBED)])

        return carry

    lax.fori_loop(0, _CPW, chunk, 0)

    @pl.when(wid == _NW - 1)
    def _():
        pltpu.sync_copy(tailflat, tailbuf)
        pltpu.sync_copy(tailbuf, tflat.at[pl.ds(_TAIL_V * _EMBED,
                                                32 * _EMBED)])


@functools.partial(
    pl.kernel,
    mesh=_mesh,
    compiler_params=pltpu.CompilerParams(use_tc_tiling_on_sc=False),
    out_type=jax.ShapeDtypeStruct((_B, _EMBED), jnp.float32),
    scratch_types=[
        pltpu.VMEM((_NCH, _CH), jnp.int32),
        pltpu.VMEM((_NCH, _CH), jnp.int32),
        pltpu.VMEM((_CH, _EMBED), jnp.float32),
        pltpu.VMEM((_CH, _EMBED), jnp.float32),
        pltpu.VMEM((_CH, _EMBED), jnp.float32),
        pltpu.VMEM((_CH, _EMBED), jnp.float32),
        pltpu.VMEM((_RPW, _EMBED), jnp.float32),
        pltpu.VMEM_SHARED((16 * 2 * _RPW, _EMBED), jnp.float32),
        pltpu.SemaphoreType.DMA,
        pltpu.SemaphoreType.DMA,
        pltpu.SemaphoreType.DMA,
        pltpu.SemaphoreType.DMA,
        pltpu.SemaphoreType.DMA,
        pltpu.SemaphoreType.DMA,
        pltpu.SemaphoreType.DMA,
        pltpu.SemaphoreType.DMA,
    ],
)
def _sc_pool(x2, table, dst3, pooled, idx_all, dst_all, rows0, rows1, rows2,
             rows3, zero_v, pooled_sh, gsem0, gsem1, gsem2, gsem3, ssem0,
             ssem1, ssem2, ssem3):
    cid = lax.axis_index("c")
    sid = lax.axis_index("s")
    wid = sid * 2 + cid

    pltpu.sync_copy(x2.at[pl.ds(wid * _NCH, _NCH)], idx_all)
    pltpu.sync_copy(dst3.at[sid], dst_all)

    z = jnp.zeros((16,), jnp.float32)

    def zero_row(r, carry):
        zero_v[r, pl.ds(0, 16)] = z
        zero_v[r, pl.ds(16, 16)] = z
        return carry

    lax.fori_loop(0, _RPW, zero_row, 0)
    pltpu.sync_copy(zero_v, pooled_sh.at[pl.ds(sid * 2 * _RPW, _RPW)])
    pltpu.sync_copy(zero_v, pooled_sh.at[pl.ds(sid * 2 * _RPW + _RPW, _RPW)])

    rows = (rows0, rows1, rows2, rows3)
    gsem = (gsem0, gsem1, gsem2, gsem3)
    ssem = (ssem0, ssem1, ssem2, ssem3)

    for j in range(3):
        pltpu.async_copy(table.at[idx_all.at[j]], rows[j], gsem[j])

    def outer(cc, carry):
        for j in range(4):
            c = cc * 4 + j
            bp = (j - 1) % 4

            def drain():
                pltpu.make_async_copy(
                    rows[bp], pooled_sh.at[dst_all.at[c]], ssem[bp]).wait()

            def prefetch():
                pltpu.async_copy(
                    table.at[idx_all.at[c + 3]], rows[bp], gsem[bp])

            if j == 0:
                pl.when(cc > 0)(drain)
                prefetch()
            elif j == 3:
                drain()
                pl.when(cc < _NCH // 4 - 1)(prefetch)
            else:
                drain()
                prefetch()
            pltpu.make_async_copy(
                table.at[idx_all.at[c]], rows[j], gsem[j]).wait()
            pltpu.async_copy(
                rows[j], pooled_sh.at[dst_all.at[c]], ssem[j], add=True)
        return carry

    lax.fori_loop(0, _NCH // 4, outer, 0)

    for c, j in ((_NCH - 2, 0), (_NCH - 1, 1)):
        pltpu.make_async_copy(
            rows[(j - 1) % 4], pooled_sh.at[dst_all.at[c]],
            ssem[(j - 1) % 4]).wait()
        pltpu.make_async_copy(
            table.at[idx_all.at[c]], rows[j], gsem[j]).wait()
        pltpu.async_copy(
            rows[j], pooled_sh.at[dst_all.at[c]], ssem[j], add=True)
    pltpu.make_async_copy(rows1, pooled_sh.at[dst_all.at[0]], ssem1).wait()

    pltpu.sync_copy(pooled_sh.at[pl.ds(sid * 2 * _RPW, _RPW)], rows0)
    pltpu.sync_copy(pooled_sh.at[pl.ds(sid * 2 * _RPW + _RPW, _RPW)], rows1)

    def merge_row(r, carry):
        rows0[r, pl.ds(0, 16)] = rows0[r, pl.ds(0, 16)] + rows1[r, pl.ds(0, 16)]
        rows0[r, pl.ds(16, 16)] = (
            rows0[r, pl.ds(16, 16)] + rows1[r, pl.ds(16, 16)])
        return carry

    lax.fori_loop(0, _RPW, merge_row, 0)
    pltpu.sync_copy(rows0, pooled.at[pl.ds(wid * _RPW, _RPW)])


def _lin_body(p_ref, w_ref, b_ref, o_ref):
    o_ref[...] = (
        jnp.dot(p_ref[...], w_ref[...], preferred_element_type=jnp.float32)
        + b_ref[...]
    )


def _linear(pooled, wt, b2):
    blk = 2048
    return pl.pallas_call(
        _lin_body,
        grid=(_B // blk,),
        in_specs=[
            pl.BlockSpec((blk, _EMBED), lambda i: (i, 0)),
            pl.BlockSpec((_EMBED, _OUT), lambda i: (0, 0)),
            pl.BlockSpec((1, _OUT), lambda i: (0, 0)),
        ],
        out_specs=pl.BlockSpec((blk, _OUT), lambda i: (i, 0)),
        out_shape=jax.ShapeDtypeStruct((_B, _OUT), jnp.float32),
    )(pooled, wt, b2)


def kernel(x, table, W, b):
    x2 = x.reshape(_NW * _NCH, _CH)
    tailflat = table[_TAIL_V:].reshape(-1)
    tflat = _sc_relayout(table.T, tailflat)
    pooled = _sc_pool(x2, tflat.reshape(_VPAD, _EMBED), _DST3)
    return _linear(pooled, W.T, b.reshape(1, _OUT))


# speedup vs baseline: 1.0982x; 1.0982x over previous
import functools

import numpy as np
import jax
import jax.numpy as jnp
from jax import lax
from jax.experimental import pallas as pl
from jax.experimental.pallas import tpu as pltpu
from jax.experimental.pallas import tpu_sc as plsc

_VOCAB, _EMBED, _OUT = 100000, 32, 128
_B, _L = 4096, 50
_NW = 32
_CH = 128
_RPW = _B // _NW
_IPW = _RPW * _L
_NCH = _IPW // _CH

_DST3 = (
    (np.arange(_IPW, dtype=np.int32) // _L).reshape(1, _NCH, _CH)
    + (np.arange(16, dtype=np.int32) * 2 * _RPW).reshape(16, 1, 1)
    + (np.arange(_NCH, dtype=np.int32) % 2 * _RPW).reshape(1, _NCH, 1)
)

_mesh = plsc.VectorSubcoreMesh(core_axis_name="c", subcore_axis_name="s")

_VPAD = 100096
_NCHUNK = 781
_TAIL_V = _NCHUNK * 128
_CPW = 25


@functools.partial(
    pl.kernel,
    mesh=_mesh,
    compiler_params=pltpu.CompilerParams(
        use_tc_tiling_on_sc=True, needs_layout_passes=False),
    out_type=jax.ShapeDtypeStruct((_VPAD * _EMBED,), jnp.float32),
    scratch_types=[
        pltpu.VMEM((_EMBED, 128), jnp.float32),
        pltpu.VMEM((128 * 33,), jnp.float32),
        pltpu.VMEM((128 * _EMBED,), jnp.float32),
        pltpu.VMEM((32 * _EMBED,), jnp.float32),
    ],
)
def _sc_relayout(tableT, tailflat, tflat, tbuf, obuf_p, obuf, tailbuf):
    cid = lax.axis_index("c")
    sid = lax.axis_index("s")
    wid = sid * 2 + cid

    lanes = lax.iota(jnp.int32, 16)
    sidx33 = lanes * 33

    def chunk(k, carry):
        j = wid + k * _NW

        @pl.when(j < _NCHUNK)
        def _():
            pltpu.sync_copy(tableT.at[:, pl.ds(j * 128, 128)], tbuf)
            for e in range(_EMBED):
                pe = sidx33 + e
                for g in range(8):
                    vec = tbuf[e, pl.ds(16 * g, 16)]
                    plsc.store_scatter(obuf_p, [pe + (16 * g * 33)], vec)

            def compact(t, carry2):
                base = lanes + t * 128
                for s in range(8):
                    a = base + 16 * s
                    vec = plsc.load_gather(
                        obuf_p, [a + lax.shift_right_logical(a, 5)])
                    obuf[pl.ds(t * 128 + 16 * s, 16)] = vec
                return carry2

            lax.fori_loop(0, 32, compact, 0)
            pltpu.sync_copy(obuf, tflat.at[pl.ds(j * 128 * _EMBED,
                                                 128 * _EMBED)])

        return carry

    lax.fori_loop(0, _CPW, chunk, 0)

    @pl.when(wid == _NW - 1)
    def _():
        pltpu.sync_copy(tailflat, tailbuf)
        pltpu.sync_copy(tailbuf, tflat.at[pl.ds(_TAIL_V * _EMBED,
                                                32 * _EMBED)])


@functools.partial(
    pl.kernel,
    mesh=_mesh,
    compiler_params=pltpu.CompilerParams(use_tc_tiling_on_sc=False),
    out_type=jax.ShapeDtypeStruct((_B, _EMBED), jnp.float32),
    scratch_types=[
        pltpu.VMEM((_NCH, _CH), jnp.int32),
        pltpu.VMEM((_NCH, _CH), jnp.int32),
        pltpu.VMEM((_CH, _EMBED), jnp.float32),
        pltpu.VMEM((_CH, _EMBED), jnp.float32),
        pltpu.VMEM((_CH, _EMBED), jnp.float32),
        pltpu.VMEM((_CH, _EMBED), jnp.float32),
        pltpu.VMEM((_RPW, _EMBED), jnp.float32),
        pltpu.VMEM_SHARED((16 * 2 * _RPW, _EMBED), jnp.float32),
        pltpu.SemaphoreType.DMA,
        pltpu.SemaphoreType.DMA,
        pltpu.SemaphoreType.DMA,
        pltpu.SemaphoreType.DMA,
        pltpu.SemaphoreType.DMA,
        pltpu.SemaphoreType.DMA,
        pltpu.SemaphoreType.DMA,
        pltpu.SemaphoreType.DMA,
    ],
)
def _sc_pool(x2, table, dst3, pooled, idx_all, dst_all, rows0, rows1, rows2,
             rows3, zero_v, pooled_sh, gsem0, gsem1, gsem2, gsem3, ssem0,
             ssem1, ssem2, ssem3):
    cid = lax.axis_index("c")
    sid = lax.axis_index("s")
    wid = sid * 2 + cid

    pltpu.sync_copy(x2.at[pl.ds(wid * _NCH, _NCH)], idx_all)
    pltpu.sync_copy(dst3.at[sid], dst_all)

    z = jnp.zeros((16,), jnp.float32)

    def zero_row(r, carry):
        zero_v[r, pl.ds(0, 16)] = z
        zero_v[r, pl.ds(16, 16)] = z
        return carry

    lax.fori_loop(0, _RPW, zero_row, 0)
    pltpu.sync_copy(zero_v, pooled_sh.at[pl.ds(sid * 2 * _RPW, _RPW)])
    pltpu.sync_copy(zero_v, pooled_sh.at[pl.ds(sid * 2 * _RPW + _RPW, _RPW)])

    rows = (rows0, rows1, rows2, rows3)
    gsem = (gsem0, gsem1, gsem2, gsem3)
    ssem = (ssem0, ssem1, ssem2, ssem3)

    for j in range(3):
        pltpu.async_copy(table.at[idx_all.at[j]], rows[j], gsem[j])

    def outer(cc, carry):
        for j in range(4):
            c = cc * 4 + j
            bp = (j - 1) % 4

            def drain():
                pltpu.make_async_copy(
                    rows[bp], pooled_sh.at[dst_all.at[c]], ssem[bp]).wait()

            def prefetch():
                pltpu.async_copy(
                    table.at[idx_all.at[c + 3]], rows[bp], gsem[bp])

            if j == 0:
                pl.when(cc > 0)(drain)
                prefetch()
            elif j == 3:
                drain()
                pl.when(cc < _NCH // 4 - 1)(prefetch)
            else:
                drain()
                prefetch()
            pltpu.make_async_copy(
                table.at[idx_all.at[c]], rows[j], gsem[j]).wait()
            pltpu.async_copy(
                rows[j], pooled_sh.at[dst_all.at[c]], ssem[j], add=True)
        return carry

    lax.fori_loop(0, _NCH // 4, outer, 0)

    for c, j in ((_NCH - 2, 0), (_NCH - 1, 1)):
        pltpu.make_async_copy(
            rows[(j - 1) % 4], pooled_sh.at[dst_all.at[c]],
            ssem[(j - 1) % 4]).wait()
        pltpu.make_async_copy(
            table.at[idx_all.at[c]], rows[j], gsem[j]).wait()
        pltpu.async_copy(
            rows[j], pooled_sh.at[dst_all.at[c]], ssem[j], add=True)
    pltpu.make_async_copy(rows1, pooled_sh.at[dst_all.at[0]], ssem1).wait()

    pltpu.sync_copy(pooled_sh.at[pl.ds(sid * 2 * _RPW, _RPW)], rows0)
    pltpu.sync_copy(pooled_sh.at[pl.ds(sid * 2 * _RPW + _RPW, _RPW)], rows1)

    def merge_row(r, carry):
        rows0[r, pl.ds(0, 16)] = rows0[r, pl.ds(0, 16)] + rows1[r, pl.ds(0, 16)]
        rows0[r, pl.ds(16, 16)] = (
            rows0[r, pl.ds(16, 16)] + rows1[r, pl.ds(16, 16)])
        return carry

    lax.fori_loop(0, _RPW, merge_row, 0)
    pltpu.sync_copy(rows0, pooled.at[pl.ds(wid * _RPW, _RPW)])


def _lin_body(p_ref, w_ref, b_ref, o_ref):
    o_ref[...] = (
        jnp.dot(p_ref[...], w_ref[...], preferred_element_type=jnp.float32)
        + b_ref[...]
    )


def _linear(pooled, wt, b2):
    blk = 2048
    return pl.pallas_call(
        _lin_body,
        grid=(_B // blk,),
        in_specs=[
            pl.BlockSpec((blk, _EMBED), lambda i: (i, 0)),
            pl.BlockSpec((_EMBED, _OUT), lambda i: (0, 0)),
            pl.BlockSpec((1, _OUT), lambda i: (0, 0)),
        ],
        out_specs=pl.BlockSpec((blk, _OUT), lambda i: (i, 0)),
        out_shape=jax.ShapeDtypeStruct((_B, _OUT), jnp.float32),
    )(pooled, wt, b2)


def kernel(x, table, W, b):
    x2 = x.reshape(_NW * _NCH, _CH)
    tailflat = table[_TAIL_V:].reshape(-1)
    tflat = _sc_relayout(table.T, tailflat)
    pooled = _sc_pool(x2, tflat.reshape(_VPAD, _EMBED), _DST3)
    return _linear(pooled, W.T, b.reshape(1, _OUT))


# speedup vs baseline: 1.5002x; 1.3660x over previous
import functools

import numpy as np
import jax
import jax.numpy as jnp
from jax import lax
from jax.experimental import pallas as pl
from jax.experimental.pallas import tpu as pltpu
from jax.experimental.pallas import tpu_sc as plsc

_VOCAB, _EMBED, _OUT = 100000, 32, 128
_B, _L = 4096, 50
_NW = 32
_CH = 128
_RPW = _B // _NW
_IPW = _RPW * _L
_NCH = _IPW // _CH

_DST3 = (
    (np.arange(_IPW, dtype=np.int32) // _L).reshape(1, _NCH, _CH)
    + (np.arange(16, dtype=np.int32) * 2 * _RPW).reshape(16, 1, 1)
    + (np.arange(_NCH, dtype=np.int32) % 2 * _RPW).reshape(1, _NCH, 1)
)

_mesh = plsc.VectorSubcoreMesh(core_axis_name="c", subcore_axis_name="s")

_VPAD = 100096
_NCHUNK = 781
_TAIL_V = _NCHUNK * 128
_CPW = 25


@functools.partial(
    pl.kernel,
    mesh=_mesh,
    compiler_params=pltpu.CompilerParams(
        use_tc_tiling_on_sc=True, needs_layout_passes=False),
    out_type=jax.ShapeDtypeStruct((_VPAD * _EMBED,), jnp.float32),
    scratch_types=[
        pltpu.VMEM((_EMBED, 128), jnp.float32),
        pltpu.VMEM((128 * 33,), jnp.float32),
        pltpu.VMEM((128 * _EMBED,), jnp.float32),
        pltpu.VMEM((32 * _EMBED,), jnp.float32),
    ],
)
def _sc_relayout(tableT, tailflat, tflat, tbuf, obuf_p, obuf, tailbuf):
    cid = lax.axis_index("c")
    sid = lax.axis_index("s")
    wid = sid * 2 + cid

    lanes = lax.iota(jnp.int32, 16)
    sidx33 = lanes * 33

    def chunk(k, carry):
        j = wid + k * _NW

        @pl.when(j < _NCHUNK)
        def _():
            pltpu.sync_copy(tableT.at[:, pl.ds(j * 128, 128)], tbuf)
            for e in range(_EMBED):
                pe = sidx33 + e
                for g in range(8):
                    vec = tbuf[e, pl.ds(16 * g, 16)]
                    plsc.store_scatter(obuf_p, [pe + (16 * g * 33)], vec)

            def compact(t, carry2):
                base = lanes + t * 128
                for s in range(8):
                    a = base + 16 * s
                    vec = plsc.load_gather(
                        obuf_p, [a + lax.shift_right_logical(a, 5)])
                    obuf[pl.ds(t * 128 + 16 * s, 16)] = vec
                return carry2

            lax.fori_loop(0, 32, compact, 0)
            pltpu.sync_copy(obuf, tflat.at[pl.ds(j * 128 * _EMBED,
                                                 128 * _EMBED)])

        return carry

    lax.fori_loop(0, _CPW, chunk, 0)

    @pl.when(wid == _NW - 1)
    def _():
        pltpu.sync_copy(tailflat, tailbuf)
        pltpu.sync_copy(tailbuf, tflat.at[pl.ds(_TAIL_V * _EMBED,
                                                32 * _EMBED)])


@functools.partial(
    pl.kernel,
    mesh=_mesh,
    compiler_params=pltpu.CompilerParams(use_tc_tiling_on_sc=False),
    out_type=jax.ShapeDtypeStruct((_B, _EMBED), jnp.float32),
    scratch_types=[
        pltpu.VMEM((_NCH, _CH), jnp.int32),
        pltpu.VMEM((_NCH, _CH), jnp.int32),
        pltpu.VMEM((_CH, _EMBED), jnp.float32),
        pltpu.VMEM((_CH, _EMBED), jnp.float32),
        pltpu.VMEM((_CH, _EMBED), jnp.float32),
        pltpu.VMEM((_CH, _EMBED), jnp.float32),
        pltpu.VMEM((_RPW, _EMBED), jnp.float32),
        pltpu.VMEM_SHARED((16 * 2 * _RPW, _EMBED), jnp.float32),
        pltpu.SemaphoreType.DMA,
        pltpu.SemaphoreType.DMA,
        pltpu.SemaphoreType.DMA,
        pltpu.SemaphoreType.DMA,
        pltpu.SemaphoreType.DMA,
        pltpu.SemaphoreType.DMA,
        pltpu.SemaphoreType.DMA,
        pltpu.SemaphoreType.DMA,
    ],
)
def _sc_pool(x2, table, dst3, pooled, idx_all, dst_all, rows0, rows1, rows2,
             rows3, zero_v, pooled_sh, gsem0, gsem1, gsem2, gsem3, ssem0,
             ssem1, ssem2, ssem3):
    cid = lax.axis_index("c")
    sid = lax.axis_index("s")
    wid = sid * 2 + cid

    pltpu.sync_copy(x2.at[pl.ds(wid * _NCH, _NCH)], idx_all)
    pltpu.sync_copy(dst3.at[sid], dst_all)

    z = jnp.zeros((16,), jnp.float32)

    def zero_row(r, carry):
        zero_v[r, pl.ds(0, 16)] = z
        zero_v[r, pl.ds(16, 16)] = z
        return carry

    lax.fori_loop(0, _RPW, zero_row, 0)
    pltpu.sync_copy(zero_v, pooled_sh.at[pl.ds(sid * 2 * _RPW, _RPW)])
    pltpu.sync_copy(zero_v, pooled_sh.at[pl.ds(sid * 2 * _RPW + _RPW, _RPW)])

    rows = (rows0, rows1, rows2, rows3)
    gsem = (gsem0, gsem1, gsem2, gsem3)
    ssem = (ssem0, ssem1, ssem2, ssem3)

    for j in range(3):
        pltpu.async_copy(table.at[idx_all.at[j]], rows[j], gsem[j])

    def outer(cc, carry):
        for j in range(4):
            c = cc * 4 + j
            bp = (j - 1) % 4

            def drain():
                pltpu.make_async_copy(
                    rows[bp], pooled_sh.at[dst_all.at[c]], ssem[bp]).wait()

            def prefetch():
                pltpu.async_copy(
                    table.at[idx_all.at[c + 3]], rows[bp], gsem[bp])

            if j == 0:
                pl.when(cc > 0)(drain)
                prefetch()
            elif j == 3:
                drain()
                pl.when(cc < _NCH // 4 - 1)(prefetch)
            else:
                drain()
                prefetch()
            pltpu.make_async_copy(
                table.at[idx_all.at[c]], rows[j], gsem[j]).wait()
            pltpu.async_copy(
                rows[j], pooled_sh.at[dst_all.at[c]], ssem[j], add=True)
        return carry

    lax.fori_loop(0, _NCH // 4, outer, 0)

    for c, j in ((_NCH - 2, 0), (_NCH - 1, 1)):
        pltpu.make_async_copy(
            rows[(j - 1) % 4], pooled_sh.at[dst_all.at[c]],
            ssem[(j - 1) % 4]).wait()
        pltpu.make_async_copy(
            table.at[idx_all.at[c]], rows[j], gsem[j]).wait()
        pltpu.async_copy(
            rows[j], pooled_sh.at[dst_all.at[c]], ssem[j], add=True)
    pltpu.make_async_copy(rows1, pooled_sh.at[dst_all.at[0]], ssem1).wait()

    pltpu.sync_copy(pooled_sh.at[pl.ds(sid * 2 * _RPW, _RPW)], rows0)
    pltpu.sync_copy(pooled_sh.at[pl.ds(sid * 2 * _RPW + _RPW, _RPW)], rows1)

    def merge_row(r, carry):
        rows0[r, pl.ds(0, 16)] = rows0[r, pl.ds(0, 16)] + rows1[r, pl.ds(0, 16)]
        rows0[r, pl.ds(16, 16)] = (
            rows0[r, pl.ds(16, 16)] + rows1[r, pl.ds(16, 16)])
        return carry

    lax.fori_loop(0, _RPW, merge_row, 0)
    pltpu.sync_copy(rows0, pooled.at[pl.ds(wid * _RPW, _RPW)])


def _lin_body(p_ref, w_ref, b_ref, o_ref):
    o_ref[...] = (
        jnp.dot(p_ref[...], w_ref[...], preferred_element_type=jnp.float32)
        + b_ref[...]
    )


def _linear(pooled, wt, b2):
    blk = 2048
    return pl.pallas_call(
        _lin_body,
        grid=(_B // blk,),
        in_specs=[
            pl.BlockSpec((blk, _EMBED), lambda i: (i, 0)),
            pl.BlockSpec((_EMBED, _OUT), lambda i: (0, 0)),
            pl.BlockSpec((1, _OUT), lambda i: (0, 0)),
        ],
        out_specs=pl.BlockSpec((blk, _OUT), lambda i: (i, 0)),
        out_shape=jax.ShapeDtypeStruct((_B, _OUT), jnp.float32),
    )(pooled, wt, b2)


def kernel(x, table, W, b):
    t4 = jnp.pad(table, ((0, 0), (0, 128 - _EMBED))).reshape(4 * _VOCAB,
                                                             _EMBED)
    x2 = (x * 4).reshape(_NW * _NCH, _CH)
    pooled = _sc_pool(x2, t4, _DST3)
    return _linear(pooled, W.T, b.reshape(1, _OUT))
